# Initial kernel scaffold; baseline (speedup 1.0000x reference)
import functools

import jax
import jax.numpy as jnp
from jax import lax
from jax.experimental import pallas as pl
from jax.experimental.pallas import tpu as pltpu
from jax.experimental.pallas import tpu_sc as plsc

N = 10000
E = 320000
NF = 128
EF = 16
H = 128
HC = 64
BOTTLE = 4
L = 7
G = 64
T = 10
EPS = 1e-7

NSUB = 16
B = 128
BLOCKS_PER_TILE = -(-E // (NSUB * B))
EPT = BLOCKS_PER_TILE * B
E_PAD = EPT * NSUB
N_ACC = 10240
ROWS_PER_TILE_OUT = N // NSUB
DRAIN_CHUNK = 125


def _make_mesh():
  try:
    return plsc.VectorSubcoreMesh(core_axis_name="c", subcore_axis_name="s")
  except Exception:
    m = plsc.VectorSubcoreMesh.__new__(plsc.VectorSubcoreMesh)
    object.__setattr__(m, "core_axis_name", "c")
    object.__setattr__(m, "subcore_axis_name", "s")
    object.__setattr__(m, "num_cores", 2)
    object.__setattr__(m, "num_subcores", NSUB)
    return m


def _sc_msgpass(hxcat, src2, dst, e_emb, wlocal2, blocal2, tvec):
  mesh = _make_mesh()

  @functools.partial(
      pl.kernel,
      mesh=mesh,
      out_type=jax.ShapeDtypeStruct((2, N, HC), jnp.float32),
      scratch_types=[
          pltpu.VMEM((B,), jnp.int32),
          pltpu.VMEM((B,), jnp.int32),
          pltpu.VMEM((B, BOTTLE), jnp.float32),
          pltpu.VMEM((B, HC), jnp.float32),
          pltpu.VMEM((B, H), jnp.float32),
          pltpu.VMEM((BOTTLE, HC), jnp.float32),
          pltpu.VMEM((HC,), jnp.float32),
          pltpu.VMEM((16,), jnp.float32),
          pltpu.VMEM_SHARED((N_ACC, H), jnp.float32),
          pltpu.SemaphoreType.DMA,
      ],
  )
  def k(hx_hbm, src_hbm, dst_hbm, e_hbm, wl_hbm, bl_hbm, t_hbm, out_hbm,
        src_v, dst_v, e_v, xrow_v, contrib_v, wl_v, bl_v, t_v, acc_sh, sem):
    c = lax.axis_index("c")
    s = lax.axis_index("s")

    pltpu.sync_copy(wl_hbm.at[c], wl_v)
    pltpu.sync_copy(bl_hbm.at[c], bl_v)
    pltpu.sync_copy(t_hbm, t_v)
    wv = [[wl_v[kk, pl.ds(v * 16, 16)] for v in range(HC // 16)]
          for kk in range(BOTTLE)]
    blv = [bl_v[pl.ds(v * 16, 16)] for v in range(HC // 16)]
    tv = t_v[pl.ds(0, 16)]

    def zero_body(r, carry):
      for v in range(H // 16):
        contrib_v[r, pl.ds(v * 16, 16)] = jnp.zeros((16,), jnp.float32)
      return carry
    lax.fori_loop(0, B, zero_body, 0)
    for i in range(N_ACC // NSUB // B):
      pltpu.sync_copy(contrib_v, acc_sh.at[pl.ds(s * (N_ACC // NSUB) + i * B, B)])
    plsc.subcore_barrier()

    def block_body(b, carry):
      base = s * EPT + b * B
      pltpu.sync_copy(src_hbm.at[c, pl.ds(base, B)], src_v)
      pltpu.sync_copy(dst_hbm.at[pl.ds(base, B)], dst_v)
      pltpu.sync_copy(e_hbm.at[pl.ds(base, B), :], e_v)
      pltpu.async_copy(hx_hbm.at[src_v], xrow_v, sem).wait()

      def edge_body(j, inner):
        e0 = e_v[j, 0]
        e1 = e_v[j, 1]
        e2 = e_v[j, 2]
        e3 = e_v[j, 3]
        for v in range(HC // 16):
          xv = xrow_v[j, pl.ds(v * 16, 16)]
          ev = (wv[0][v] * e0 + wv[1][v] * e1 + wv[2][v] * e2
                + wv[3][v] * e3 + blv[v])
          m = jnp.maximum(xv + ev, 0.0) + EPS
          w = jnp.exp(m * tv)
          contrib_v[j, pl.ds(v * 16, 16)] = w * m
          contrib_v[j, pl.ds(HC + v * 16, 16)] = w
        return inner
      lax.fori_loop(0, B, edge_body, 0)

      pltpu.sync_copy(contrib_v, acc_sh.at[dst_v], add=True)
      return carry
    lax.fori_loop(0, BLOCKS_PER_TILE, block_body, 0)

    plsc.subcore_barrier()

    for ch in range(ROWS_PER_TILE_OUT // DRAIN_CHUNK):
      r0 = s * ROWS_PER_TILE_OUT + ch * DRAIN_CHUNK
      pltpu.sync_copy(acc_sh.at[pl.ds(r0, DRAIN_CHUNK)],
                      contrib_v.at[pl.ds(0, DRAIN_CHUNK)])

      def drain_body(r, carry):
        for v in range(HC // 16):
          num = contrib_v[r, pl.ds(v * 16, 16)]
          den = contrib_v[r, pl.ds(HC + v * 16, 16)]
          xrow_v[r, pl.ds(v * 16, 16)] = num / (den + 1e-16)
        return carry
      lax.fori_loop(0, DRAIN_CHUNK, drain_body, 0)
      pltpu.sync_copy(xrow_v.at[pl.ds(0, DRAIN_CHUNK)],
                      out_hbm.at[c, pl.ds(r0, DRAIN_CHUNK)])

  return k(hxcat, src2, dst, e_emb, wlocal2, blocal2, tvec)


def _tc_encode_body(x_ref, w_ref, b_ref, h0_ref, hx2_ref):
  h0 = jnp.dot(x_ref[...], w_ref[...],
               preferred_element_type=jnp.float32) + b_ref[...]
  h0_ref[...] = h0
  hx2_ref[0] = h0[:, :HC]
  hx2_ref[1] = h0[:, HC:]


def _tc_edge_body(ea_ref, wg_ref, bg_ref, out_ref):
  out_ref[...] = jnp.dot(ea_ref[...], wg_ref[...],
                         preferred_element_type=jnp.float32) + bg_ref[...]


def _tc_layer_body(has_res, last, hrun_ref, hx2_ref, agg2_ref, wm_ref, bm_ref,
                   g_ref, bt_ref, *rest):
  hx = jnp.concatenate([hx2_ref[0], hx2_ref[1]], axis=1)
  agg = jnp.concatenate([agg2_ref[0], agg2_ref[1]], axis=1)
  hres = hx + agg
  gout = jnp.dot(hres, wm_ref[...],
                 preferred_element_type=jnp.float32) + bm_ref[...]
  if has_res:
    h = gout + hrun_ref[...]
  else:
    h = gout
  mu = jnp.mean(h, axis=0, keepdims=True)
  var = jnp.mean((h - mu) ** 2, axis=0, keepdims=True)
  h1 = (h - mu) / jnp.sqrt(var + 1e-5) * g_ref[...] + bt_ref[...]
  if last:
    batch_ref, wp_ref, bp_ref, out_ref = rest
    onehot = (batch_ref[...] == lax.broadcasted_iota(
        jnp.int32, (N, G), 1)).astype(jnp.float32)
    hg = lax.dot_general(onehot, h1, (((0,), (0,)), ((), ())),
                         preferred_element_type=jnp.float32)
    cnt = lax.dot_general(onehot, jnp.ones((N, 1), jnp.float32),
                          (((0,), (0,)), ((), ())),
                          preferred_element_type=jnp.float32)
    hg = hg / jnp.maximum(cnt, 1.0)
    out_ref[...] = jnp.dot(hg, wp_ref[...],
                           preferred_element_type=jnp.float32) + bp_ref[...]
  else:
    hnew_ref, hx2o_ref = rest
    h2 = jnp.maximum(h1, 0.0)
    hnew_ref[...] = h
    hx2o_ref[0] = h2[:, :HC]
    hx2o_ref[1] = h2[:, HC:]


def kernel(x, edge_index, edge_attr, batch, W_in, b_in, W_glob, b_glob,
           W_local, b_local, t, W_mlp, b_mlp, gamma, beta, W_pred, b_pred):
  src = edge_index[0].astype(jnp.int32)
  dst = edge_index[1].astype(jnp.int32)

  pad = E_PAD - E
  src_p = jnp.pad(src, (0, pad))
  dst_p = jnp.pad(dst, (0, pad), constant_values=N)
  src2 = jnp.stack([src_p, src_p + N])
  ea_p = jnp.pad(edge_attr, ((0, pad), (0, 0)))
  wlocal2 = jnp.stack([W_local[:, :, :HC], W_local[:, :, HC:]], axis=1)
  blocal2 = jnp.stack([b_local[:, :HC], b_local[:, HC:]], axis=1)
  batch2d = batch.astype(jnp.int32)[:, None]

  h0, hx2 = pl.pallas_call(
      _tc_encode_body,
      out_shape=(jax.ShapeDtypeStruct((N, H), jnp.float32),
                 jax.ShapeDtypeStruct((2, N, HC), jnp.float32)),
  )(x, W_in, b_in[None, :])

  e_emb = pl.pallas_call(
      _tc_edge_body,
      out_shape=jax.ShapeDtypeStruct((E_PAD, BOTTLE), jnp.float32),
  )(ea_p, W_glob, b_glob[None, :])

  hxcat = hx2.reshape(2 * N, HC)
  hrun = h0

  for l in range(L):
    tvec = jnp.full((16,), t[l], jnp.float32)
    agg2 = _sc_msgpass(hxcat, src2, dst_p, e_emb,
                       wlocal2[l], blocal2[l], tvec)
    last = (l == L - 1)
    if not last:
      outs = (jax.ShapeDtypeStruct((N, H), jnp.float32),
              jax.ShapeDtypeStruct((2, N, HC), jnp.float32))
    else:
      outs = jax.ShapeDtypeStruct((G, T), jnp.float32)
    body = functools.partial(_tc_layer_body, l > 0, last)
    args = [hrun, hx2, agg2, W_mlp[l], b_mlp[l][None, :],
            gamma[l][None, :], beta[l][None, :]]
    if last:
      args += [batch2d, W_pred, b_pred[None, :]]
    res = pl.pallas_call(
        body, out_shape=outs,
    )(*args)
    if not last:
      hrun, hx2 = res
      hxcat = hx2.reshape(2 * N, HC)
    else:
      return res


# speedup vs baseline: 2.1948x; 2.1948x over previous
import functools

import jax
import jax.numpy as jnp
from jax import lax
from jax.experimental import pallas as pl
from jax.experimental.pallas import tpu as pltpu
from jax.experimental.pallas import tpu_sc as plsc

N = 10000
E = 320000
NF = 128
EF = 16
H = 128
HC = 64
BOTTLE = 4
L = 7
G = 64
T = 10
EPS = 1e-7

NSUB = 16
B = 128
BLOCKS_PER_TILE = -(-E // (NSUB * B))
EPT = BLOCKS_PER_TILE * B
E_PAD = EPT * NSUB
N_ACC = 10016
INIT_STRIPE = N_ACC // NSUB
DRAIN_BASE = 624
DRAIN_CHUNK = 128


def _make_mesh():
  try:
    return plsc.VectorSubcoreMesh(core_axis_name="c", subcore_axis_name="s")
  except Exception:
    m = plsc.VectorSubcoreMesh.__new__(plsc.VectorSubcoreMesh)
    object.__setattr__(m, "core_axis_name", "c")
    object.__setattr__(m, "subcore_axis_name", "s")
    object.__setattr__(m, "num_cores", 2)
    object.__setattr__(m, "num_subcores", NSUB)
    return m


def _sc_msgpass(hx, src, dst, e_emb, wlocal2, blocal2, tvec):
  mesh = _make_mesh()

  @functools.partial(
      pl.kernel,
      mesh=mesh,
      out_type=jax.ShapeDtypeStruct((2, N, HC), jnp.float32),
      scratch_types=[
          pltpu.VMEM((B,), jnp.int32),
          pltpu.VMEM((B,), jnp.int32),
          pltpu.VMEM((B * BOTTLE + 16,), jnp.float32),
          pltpu.VMEM((B, H), jnp.float32),
          pltpu.VMEM((B, H), jnp.float32),
          pltpu.VMEM((BOTTLE, HC), jnp.float32),
          pltpu.VMEM((HC,), jnp.float32),
          pltpu.VMEM((16,), jnp.float32),
          pltpu.VMEM((DRAIN_CHUNK, HC), jnp.float32),
          pltpu.VMEM_SHARED((N_ACC, H), jnp.float32),
          pltpu.SemaphoreType.DMA,
      ],
  )
  def k(hx_hbm, src_hbm, dst_hbm, e_hbm, wl_hbm, bl_hbm, t_hbm, out_hbm,
        src_v, dst_v, e_v, xrow_v, contrib_v, wl_v, bl_v, t_v, agg_v,
        acc_sh, sem):
    c = lax.axis_index("c")
    s = lax.axis_index("s")
    cbase = c * HC

    pltpu.sync_copy(wl_hbm.at[c], wl_v)
    pltpu.sync_copy(bl_hbm.at[c], bl_v)
    pltpu.sync_copy(t_hbm, t_v)
    wv = [[wl_v[kk, pl.ds(v * 16, 16)] for v in range(HC // 16)]
          for kk in range(BOTTLE)]
    blv = [bl_v[pl.ds(v * 16, 16)] for v in range(HC // 16)]
    tv = t_v[pl.ds(0, 16)]

    def zero_body(r, carry):
      for v in range(H // 16):
        contrib_v[r, pl.ds(v * 16, 16)] = jnp.zeros((16,), jnp.float32)
      return carry
    lax.fori_loop(0, B, zero_body, 0)
    for off, sz in ((0, 128), (128, 128), (256, 128), (384, 128), (512, 114)):
      pltpu.sync_copy(contrib_v.at[pl.ds(0, sz)],
                      acc_sh.at[pl.ds(s * INIT_STRIPE + off, sz)])
    plsc.subcore_barrier()

    def block_body(b, carry):
      base = s * EPT + b * B
      pltpu.sync_copy(src_hbm.at[pl.ds(base, B)], src_v)
      pltpu.sync_copy(dst_hbm.at[pl.ds(base, B)], dst_v)
      pltpu.sync_copy(e_hbm.at[pl.ds(base * BOTTLE, B * BOTTLE)],
                      e_v.at[pl.ds(0, B * BOTTLE)])
      pltpu.async_copy(hx_hbm.at[src_v], xrow_v, sem).wait()

      def edge_body(j, inner):
        evec = e_v[pl.ds(j * BOTTLE, 16)]
        e0 = evec[0]
        e1 = evec[1]
        e2 = evec[2]
        e3 = evec[3]
        for v in range(HC // 16):
          xv = xrow_v[j, pl.ds(cbase + v * 16, 16)]
          ev = (wv[0][v] * e0 + wv[1][v] * e1 + wv[2][v] * e2
                + wv[3][v] * e3 + blv[v])
          m = jnp.maximum(xv + ev, 0.0) + EPS
          w = jnp.exp(m * tv)
          contrib_v[j, pl.ds(v * 16, 16)] = w * m
          contrib_v[j, pl.ds(HC + v * 16, 16)] = w
        return inner
      lax.fori_loop(0, B, edge_body, 0)

      pltpu.sync_copy(contrib_v, acc_sh.at[dst_v], add=True)
      return carry
    lax.fori_loop(0, BLOCKS_PER_TILE, block_body, 0)

    plsc.subcore_barrier()

    def drain_chunk(r0, sz):
      pltpu.sync_copy(acc_sh.at[pl.ds(r0, sz)], contrib_v.at[pl.ds(0, sz)])

      def drain_body(r, carry):
        for v in range(HC // 16):
          num = contrib_v[r, pl.ds(v * 16, 16)]
          den = contrib_v[r, pl.ds(HC + v * 16, 16)]
          agg_v[r, pl.ds(v * 16, 16)] = num / (den + 1e-16)
        return carry
      lax.fori_loop(0, sz, drain_body, 0)
      pltpu.sync_copy(agg_v.at[pl.ds(0, sz)], out_hbm.at[c, pl.ds(r0, sz)])

    for off, sz in ((0, 128), (128, 128), (256, 128), (384, 128), (512, 112)):
      drain_chunk(s * DRAIN_BASE + off, sz)

    @pl.when(s == NSUB - 1)
    def _():
      drain_chunk(NSUB * DRAIN_BASE, N - NSUB * DRAIN_BASE)

  return k(hx, src, dst, e_emb, wlocal2, blocal2, tvec)


def _tc_encode_body(x_ref, w_ref, b_ref, h0_ref):
  h0_ref[...] = jnp.dot(x_ref[...], w_ref[...],
                        preferred_element_type=jnp.float32) + b_ref[...]


def _tc_edge_body(ea_ref, wg_ref, bg_ref, out_ref):
  out_ref[...] = jnp.dot(ea_ref[...], wg_ref[...],
                         preferred_element_type=jnp.float32) + bg_ref[...]


def _tc_layer_body(has_res, last, hrun_ref, hx_ref, agg2_ref, wm_ref, bm_ref,
                   g_ref, bt_ref, *rest):
  hx = hx_ref[...]
  agg = jnp.concatenate([agg2_ref[0], agg2_ref[1]], axis=1)
  hres = hx + agg
  gout = jnp.dot(hres, wm_ref[...],
                 preferred_element_type=jnp.float32) + bm_ref[...]
  if has_res:
    h = gout + hrun_ref[...]
  else:
    h = gout
  mu = jnp.mean(h, axis=0, keepdims=True)
  var = jnp.mean((h - mu) ** 2, axis=0, keepdims=True)
  h1 = (h - mu) / jnp.sqrt(var + 1e-5) * g_ref[...] + bt_ref[...]
  if last:
    batch_ref, wp_ref, bp_ref, out_ref = rest
    onehot = (batch_ref[...] == lax.broadcasted_iota(
        jnp.int32, (N, G), 1)).astype(jnp.float32)
    hg = lax.dot_general(onehot, h1, (((0,), (0,)), ((), ())),
                         preferred_element_type=jnp.float32)
    cnt = lax.dot_general(onehot, jnp.ones((N, 1), jnp.float32),
                          (((0,), (0,)), ((), ())),
                          preferred_element_type=jnp.float32)
    hg = hg / jnp.maximum(cnt, 1.0)
    out_ref[...] = jnp.dot(hg, wp_ref[...],
                           preferred_element_type=jnp.float32) + bp_ref[...]
  else:
    hnew_ref, h2_ref = rest
    hnew_ref[...] = h
    h2_ref[...] = jnp.maximum(h1, 0.0)


def kernel(x, edge_index, edge_attr, batch, W_in, b_in, W_glob, b_glob,
           W_local, b_local, t, W_mlp, b_mlp, gamma, beta, W_pred, b_pred):
  src = edge_index[0].astype(jnp.int32)
  dst = edge_index[1].astype(jnp.int32)

  pad = E_PAD - E
  src_p = jnp.pad(src, (0, pad))
  dst_p = jnp.pad(dst, (0, pad), constant_values=N)
  ea_p = jnp.pad(edge_attr, ((0, pad), (0, 0)))
  wlocal2 = jnp.stack([W_local[:, :, :HC], W_local[:, :, HC:]], axis=1)
  blocal2 = jnp.stack([b_local[:, :HC], b_local[:, HC:]], axis=1)
  batch2d = batch.astype(jnp.int32)[:, None]

  h0 = pl.pallas_call(
      _tc_encode_body,
      out_shape=jax.ShapeDtypeStruct((N, H), jnp.float32),
  )(x, W_in, b_in[None, :])

  erows = E_PAD // 8
  eb = erows // 16
  wg_big = jnp.kron(jnp.eye(8, dtype=jnp.float32), W_glob)
  bg_big = jnp.tile(b_glob, 8)[None, :]
  e_emb = pl.pallas_call(
      _tc_edge_body,
      grid=(16,),
      in_specs=[pl.BlockSpec((eb, 8 * EF), lambda i: (i, 0)),
                pl.BlockSpec((8 * EF, 8 * BOTTLE), lambda i: (0, 0)),
                pl.BlockSpec((1, 8 * BOTTLE), lambda i: (0, 0))],
      out_specs=pl.BlockSpec((eb, 8 * BOTTLE), lambda i: (i, 0)),
      out_shape=jax.ShapeDtypeStruct((erows, 8 * BOTTLE), jnp.float32),
  )(ea_p.reshape(erows, 8 * EF), wg_big, bg_big)

  hx = h0
  hrun = h0

  for l in range(L):
    tvec = jnp.full((16,), t[l], jnp.float32)
    agg2 = _sc_msgpass(hx, src_p, dst_p, e_emb.reshape(E_PAD * BOTTLE),
                       wlocal2[l], blocal2[l], tvec)
    last = (l == L - 1)
    if not last:
      outs = (jax.ShapeDtypeStruct((N, H), jnp.float32),
              jax.ShapeDtypeStruct((N, H), jnp.float32))
    else:
      outs = jax.ShapeDtypeStruct((G, T), jnp.float32)
    body = functools.partial(_tc_layer_body, l > 0, last)
    args = [hrun, hx, agg2, W_mlp[l], b_mlp[l][None, :],
            gamma[l][None, :], beta[l][None, :]]
    if last:
      args += [batch2d, W_pred, b_pred[None, :]]
    res = pl.pallas_call(
        body, out_shape=outs,
    )(*args)
    if not last:
      hrun, hx = res
    else:
      return res
